# Initial kernel scaffold; baseline (speedup 1.0000x reference)
import jax
import jax.numpy as jnp
from jax.experimental import pallas as pl


def kernel(x, edge_index, W1, a_src1, a_dst1, b1, W2, a_src2, a_dst2, b2):
    raise NotImplementedError("write your pallas kernel here")


# speedup vs baseline: 36.2838x; 36.2838x over previous
import functools

import jax
import jax.numpy as jnp
from jax import lax
from jax.experimental import pallas as pl
from jax.experimental.pallas import tpu as pltpu
from jax.experimental.pallas import tpu_sc as plsc

N = 10000
E = 320000
HEADS = 2
NC, NS, L = 2, 16, 16
K = 128
CHUNKS = E // K
NPAD = 10112
ROWS_PER_TILE = NPAD // NS


def _prep_body(x_ref, w_ref, asrc_ref, adst_ref, haug_ref, asd_ref, *, o, wa):
    xb = x_ref[...]
    h = jnp.dot(xb, w_ref[...], preferred_element_type=jnp.float32,
                precision=jax.lax.Precision.HIGHEST)
    bn = xb.shape[0]
    for hd in range(HEADS):
        hh = h[:, hd * o:(hd + 1) * o]
        ones = jnp.ones((bn, 1), jnp.float32)
        pad = jnp.zeros((bn, wa - o - 1), jnp.float32)
        haug_ref[hd] = jnp.concatenate([hh, ones, pad], axis=1)
        asd_ref[0, hd, 0, :] = jnp.sum(hh * asrc_ref[hd][None, :], axis=1)
        asd_ref[0, hd, 1, :] = jnp.sum(hh * adst_ref[hd][None, :], axis=1)


def _prep(x_in, W, a_src, a_dst, o, wa, bn=400):
    cin = x_in.shape[1]
    grid = (N // bn,)
    return pl.pallas_call(
        functools.partial(_prep_body, o=o, wa=wa),
        grid=grid,
        in_specs=[
            pl.BlockSpec((bn, cin), lambda i: (i, 0)),
            pl.BlockSpec((cin, HEADS * o), lambda i: (0, 0)),
            pl.BlockSpec((HEADS, o), lambda i: (0, 0)),
            pl.BlockSpec((HEADS, o), lambda i: (0, 0)),
        ],
        out_specs=[
            pl.BlockSpec((HEADS, bn, wa), lambda i: (0, i, 0)),
            pl.BlockSpec((1, HEADS, 2, bn), lambda i: (i, 0, 0, 0)),
        ],
        out_shape=[
            jax.ShapeDtypeStruct((HEADS, N, wa), jnp.float32),
            jax.ShapeDtypeStruct((N // bn, HEADS, 2, bn), jnp.float32),
        ],
    )(x_in, W, a_src, a_dst)


def _m_body(asd_ref, m_ref):
    a = asd_ref[...]
    mx = jnp.max(a, axis=(0, 3))
    m = jnp.maximum(mx[:, 0] + mx[:, 1], 0.0)
    m_ref[...] = jnp.broadcast_to(m[:, None], (HEADS, 16))


def _m_bound(asd):
    return pl.pallas_call(
        _m_body,
        out_shape=jax.ShapeDtypeStruct((HEADS, 16), jnp.float32),
    )(asd)


def _make_edge_kernel(wa):
    mesh = plsc.VectorSubcoreMesh(core_axis_name="c", subcore_axis_name="s")

    @functools.partial(
        pl.kernel,
        mesh=mesh,
        compiler_params=pltpu.CompilerParams(
            needs_layout_passes=False, use_tc_tiling_on_sc=False),
        out_type=jax.ShapeDtypeStruct((HEADS * NPAD, wa), jnp.float32),
        scratch_types=[
            pltpu.VMEM((N,), jnp.float32),
            pltpu.VMEM((N,), jnp.float32),
            pltpu.VMEM((16,), jnp.float32),
            pltpu.VMEM((K,), jnp.int32),
            pltpu.VMEM((K,), jnp.int32),
            pltpu.VMEM((K,), jnp.int32),
            pltpu.VMEM((K,), jnp.float32),
            pltpu.VMEM((K, wa), jnp.float32),
            pltpu.VMEM_SHARED((NPAD, wa), jnp.float32),
            pltpu.SemaphoreType.DMA,
        ],
    )
    def k(haug, ei, asd, mb, zeros_hbm, out,
          as_t, ad_t, m_v, src_c, dst_c, idx_c, p_c, rows, acc, sem):
        head = lax.axis_index("c")
        sid = lax.axis_index("s")
        pltpu.sync_copy(asd.at[head, 0], as_t)
        pltpu.sync_copy(asd.at[head, 1], ad_t)
        pltpu.sync_copy(mb.at[head], m_v)
        pltpu.sync_copy(zeros_hbm.at[pl.ds(sid * ROWS_PER_TILE, ROWS_PER_TILE)],
                        acc.at[pl.ds(sid * ROWS_PER_TILE, ROWS_PER_TILE)])
        plsc.subcore_barrier()

        m_vec = m_v[...]
        off = head * N
        offp = head * NPAD
        n_iters = (CHUNKS + NS - 1) // NS

        def chunk_body(ci, carry):
            g = sid + ci * NS

            @pl.when(g < CHUNKS)
            def _():
                e0 = g * K
                pltpu.sync_copy(ei.at[0, pl.ds(e0, K)], src_c)
                pltpu.sync_copy(ei.at[1, pl.ds(e0, K)], dst_c)
                for j in range(K // L):
                    sl = pl.ds(j * L, L)
                    s_i = src_c[sl]
                    d_i = dst_c[sl]
                    a_s = plsc.load_gather(as_t, [s_i])
                    a_d = plsc.load_gather(ad_t, [d_i])
                    al = a_s + a_d
                    al = jnp.maximum(al, 0.2 * al)
                    p_c[sl] = jnp.exp(al - m_vec)
                    idx_c[sl] = s_i + off
                pltpu.async_copy(haug.at[idx_c], rows, sem).wait()

                def row_body(r, c2):
                    pr = plsc.load_gather(p_c, [jnp.full((L,), r, jnp.int32)])
                    for w in range(wa // L):
                        cw = pl.ds(w * L, L)
                        rows[r, cw] = rows[r, cw] * pr
                    return c2
                lax.fori_loop(0, K, row_body, 0)
                pltpu.sync_copy(rows, acc.at[dst_c], add=True)
            return carry

        lax.fori_loop(0, n_iters, chunk_body, 0)
        plsc.subcore_barrier()
        pltpu.sync_copy(
            acc.at[pl.ds(sid * ROWS_PER_TILE, ROWS_PER_TILE)],
            out.at[pl.ds(offp + sid * ROWS_PER_TILE, ROWS_PER_TILE)])

    return k


def _combine_body(acc_ref, haug_ref, asd_ref, m_ref, b_ref, out_ref, *, o, wa):
    for hd in range(HEADS):
        a_s = asd_ref[0, hd, 0, :]
        a_d = asd_ref[0, hd, 1, :]
        al = a_s + a_d
        al = jnp.maximum(al, 0.2 * al)
        ps = jnp.exp(al - m_ref[hd, 0])
        hh = haug_ref[hd, :, 0:o]
        num = acc_ref[hd, :, 0:o] + ps[:, None] * hh
        den = acc_ref[hd, :, o] + ps + 1e-16
        v = num / den[:, None] + b_ref[0, hd * o:(hd + 1) * o][None, :]
        out_ref[:, hd * o:(hd + 1) * o] = jnp.where(
            v > 0, v, jnp.exp(jnp.minimum(v, 0.0)) - 1.0)


def _combine(acc, haug, asd, m_b, b, o, wa, bn=400):
    grid = (N // bn,)
    return pl.pallas_call(
        functools.partial(_combine_body, o=o, wa=wa),
        grid=grid,
        in_specs=[
            pl.BlockSpec((HEADS, bn, wa), lambda i: (0, i, 0)),
            pl.BlockSpec((HEADS, bn, wa), lambda i: (0, i, 0)),
            pl.BlockSpec((1, HEADS, 2, bn), lambda i: (i, 0, 0, 0)),
            pl.BlockSpec((HEADS, 16), lambda i: (0, 0)),
            pl.BlockSpec((1, HEADS * o), lambda i: (0, 0)),
        ],
        out_specs=pl.BlockSpec((bn, HEADS * o), lambda i: (i, 0)),
        out_shape=jax.ShapeDtypeStruct((N, HEADS * o), jnp.float32),
    )(acc, haug, asd, m_b, b.reshape(1, -1))


def _gat_layer(x_in, edge_index, W, a_src, a_dst, b, o, wa):
    haug, asd_blk = _prep(x_in, W, a_src, a_dst, o, wa)
    m_b = _m_bound(asd_blk)
    asd = asd_blk.transpose(1, 2, 0, 3).reshape(HEADS, 2, N)
    zeros = jnp.zeros((NPAD, wa), jnp.float32)
    edge_k = _make_edge_kernel(wa)
    accp = edge_k(haug.reshape(HEADS * N, wa), edge_index, asd, m_b, zeros)
    accp = accp.reshape(HEADS, NPAD, wa)
    acc = accp[:, :N, :]
    return _combine(acc, haug, asd_blk, m_b, b, o, wa)


def kernel(x, edge_index, W1, a_src1, a_dst1, b1, W2, a_src2, a_dst2, b2):
    h1 = _gat_layer(x, edge_index, W1, a_src1, a_dst1, b1, 128, 144)
    h2 = _gat_layer(h1, edge_index, W2, a_src2, a_dst2, b2, 64, 80)
    return (h2, x)
